# baseline (speedup 1.0000x reference)
import jax
import jax.numpy as jnp
from jax import lax
from jax.experimental import pallas as pl
from jax.experimental.pallas import tpu as pltpu
from jax.experimental.pallas import tpu_sc as plsc

B = 1024
W = 200
NUM_INPUTS = 8
D = 64
HIST = 150
FUT = W - HIST

NC = 2
NS = 16
NW = NC * NS

CH_H, NCH_H = 120, 40
CH_F, NCH_F = 80, 20
S_PER_W = B // NW


def _sc_gather(e0_hbm, e1_hbm, idx0_hbm, idxh_hbm, idxf_hbm,
               gh_hbm, gf_hbm, s_hbm,
               idx0_v, idxh_v, idxf_v, r_a, r_b, r0_v, sem_a, sem_b, sem0):
    wid = lax.axis_index("s") * NC + lax.axis_index("c")

    pltpu.sync_copy(idx0_hbm.at[wid], idx0_v)
    pltpu.async_copy(e0_hbm.at[idx0_v.at[0]], r0_v, sem0).wait()
    pltpu.sync_copy(r0_v, s_hbm.at[pl.ds(wid * S_PER_W, S_PER_W)])

    pltpu.sync_copy(idxh_hbm.at[wid], idxh_v)
    base_h = wid * (NCH_H * CH_H)

    def body_h(k, carry):
        j0 = 2 * k
        j1 = j0 + 1
        c0 = pltpu.async_copy(e1_hbm.at[idxh_v.at[j0]], r_a, sem_a)
        c1 = pltpu.async_copy(e1_hbm.at[idxh_v.at[j1]], r_b, sem_b)
        c0.wait()
        pltpu.sync_copy(r_a, gh_hbm.at[pl.ds(base_h + j0 * CH_H, CH_H), 1])
        c1.wait()
        pltpu.sync_copy(r_b, gh_hbm.at[pl.ds(base_h + j1 * CH_H, CH_H), 1])
        return carry

    lax.fori_loop(0, NCH_H // 2, body_h, 0)

    pltpu.sync_copy(idxf_hbm.at[wid], idxf_v)
    base_f = wid * (NCH_F * CH_F)

    def body_f(k, carry):
        j0 = 2 * k
        j1 = j0 + 1
        ra = r_a.at[pl.ds(0, CH_F)]
        rb = r_b.at[pl.ds(0, CH_F)]
        c0 = pltpu.async_copy(e1_hbm.at[idxf_v.at[j0]], ra, sem_a)
        c1 = pltpu.async_copy(e1_hbm.at[idxf_v.at[j1]], rb, sem_b)
        c0.wait()
        pltpu.sync_copy(ra, gf_hbm.at[pl.ds(base_f + j0 * CH_F, CH_F), 0])
        c1.wait()
        pltpu.sync_copy(rb, gf_hbm.at[pl.ds(base_f + j1 * CH_F, CH_F), 0])
        return carry

    lax.fori_loop(0, NCH_F // 2, body_f, 0)


def _tc_body(xh_ref, xf_ref, gh_ref, gf_ref, wh_ref, bh_ref, wf_ref, bf_ref,
             hist_ref, fut_ref):
    gh = gh_ref[...]
    gf = gf_ref[...]
    low = lax.broadcasted_iota(jnp.int32, (1, 2 * D), 1) < D
    xh = xh_ref[...]
    xf = xf_ref[...]

    def d(x, i, wref, bref):
        return x[:, i:i + 1] * wref[i][None, :] + bref[i][None, :]

    hist_ref[:, 0:128] = jnp.where(low, d(xh, 7, wh_ref, bh_ref), gh)
    hist_ref[:, 128:256] = d(xh, 5, wh_ref, bh_ref) + d(xh, 6, wh_ref, bh_ref)
    hist_ref[:, 256:384] = d(xh, 2, wh_ref, bh_ref) + d(xh, 3, wh_ref, bh_ref)
    hist_ref[:, 384:448] = d(xh, 4, wh_ref, bh_ref)[:, 0:64]
    fut_ref[:, 0:128] = jnp.where(low, gf, d(xf, 5, wf_ref, bf_ref))
    fut_ref[:, 128:192] = d(xf, 6, wf_ref, bf_ref)[:, 0:64]


def kernel(inputs, E0, E1, W2, b2, W3, b3, W4, b4, W5, b5, W6, b6, W7, b7):
    f32 = jnp.float32
    idx0 = inputs[:, 0, 0].astype(jnp.int32).reshape(NW, 1, S_PER_W)
    idxh = inputs[:, :HIST, 1].astype(jnp.int32).reshape(NW, NCH_H, CH_H)
    idxf = inputs[:, HIST:, 1].astype(jnp.int32).reshape(NW, NCH_F, CH_F)

    mesh = plsc.VectorSubcoreMesh(core_axis_name="c", subcore_axis_name="s")
    sc = pl.kernel(
        _sc_gather,
        mesh=mesh,
        out_type=[
            jax.ShapeDtypeStruct((B * HIST, 2, D), f32),
            jax.ShapeDtypeStruct((B * FUT, 2, D), f32),
            jax.ShapeDtypeStruct((B, D), f32),
        ],
        scratch_types=[
            pltpu.VMEM((1, S_PER_W), jnp.int32),
            pltpu.VMEM((NCH_H, CH_H), jnp.int32),
            pltpu.VMEM((NCH_F, CH_F), jnp.int32),
            pltpu.VMEM((CH_H, D), f32),
            pltpu.VMEM((CH_H, D), f32),
            pltpu.VMEM((S_PER_W, D), f32),
            pltpu.SemaphoreType.DMA,
            pltpu.SemaphoreType.DMA,
            pltpu.SemaphoreType.DMA,
        ],
        compiler_params=pltpu.CompilerParams(use_tc_tiling_on_sc=False),
    )
    gh, gf, s_rows = sc(E0, E1, idx0, idxh, idxf)

    xh_arr = inputs[:, :HIST, :].reshape(B * HIST, NUM_INPUTS)
    xf_arr = inputs[:, HIST:, :].reshape(B * FUT, NUM_INPUTS)
    z64 = jnp.zeros((D,), f32)
    z128 = jnp.zeros((2 * D,), f32)

    def lo(v):
        return jnp.concatenate([v.reshape(D), z64])

    def hi(v):
        return jnp.concatenate([z64, v.reshape(D)])

    wh = jnp.stack([z128, z128, lo(W2), hi(W3), lo(W4), lo(W5), hi(W6), lo(W7)])
    bh = jnp.stack([z128, z128, lo(b2), hi(b3), lo(b4), lo(b5), hi(b6), lo(b7)])
    wf = jnp.stack([z128, z128, z128, z128, z128, hi(W5), lo(W6), z128])
    bf = jnp.stack([z128, z128, z128, z128, z128, hi(b5), lo(b6), z128])

    bb = 8
    bh_rows = bb * HIST
    bf_rows = bb * FUT
    hist_flat, fut_flat = pl.pallas_call(
        _tc_body,
        grid=(B // bb,),
        in_specs=[
            pl.BlockSpec((bh_rows, NUM_INPUTS), lambda b: (b, 0)),
            pl.BlockSpec((bf_rows, NUM_INPUTS), lambda b: (b, 0)),
            pl.BlockSpec((bh_rows, 2 * D), lambda b: (b, 0)),
            pl.BlockSpec((bf_rows, 2 * D), lambda b: (b, 0)),
            pl.BlockSpec((NUM_INPUTS, 2 * D), lambda b: (0, 0)),
            pl.BlockSpec((NUM_INPUTS, 2 * D), lambda b: (0, 0)),
            pl.BlockSpec((NUM_INPUTS, 2 * D), lambda b: (0, 0)),
            pl.BlockSpec((NUM_INPUTS, 2 * D), lambda b: (0, 0)),
        ],
        out_specs=[
            pl.BlockSpec((bh_rows, 7 * D), lambda b: (b, 0)),
            pl.BlockSpec((bf_rows, 3 * D), lambda b: (b, 0)),
        ],
        out_shape=[
            jax.ShapeDtypeStruct((B * HIST, 7 * D), f32),
            jax.ShapeDtypeStruct((B * FUT, 3 * D), f32),
        ],
    )(xh_arr, xf_arr, gh.reshape(B * HIST, 2 * D), gf.reshape(B * FUT, 2 * D),
      wh, bh, wf, bf)

    static = s_rows.reshape(B, 1, D)
    historical = hist_flat.reshape(B, HIST, 7, D)
    future = fut_flat.reshape(B, FUT, 3, D)
    return (static, historical, future)


# speedup vs baseline: 1.5139x; 1.5139x over previous
import jax
import jax.numpy as jnp
from jax import lax
from jax.experimental import pallas as pl
from jax.experimental.pallas import tpu as pltpu
from jax.experimental.pallas import tpu_sc as plsc

B = 1024
W = 200
NUM_INPUTS = 8
D = 64
HIST = 150
FUT = W - HIST

NC = 2
NS = 16
NW = NC * NS

CH_H, NCH_H = 120, 40
CH_F, NCH_F = 80, 20
S_PER_W = B // NW


def _sc_gather(e0_hbm, e1_hbm, idx0_hbm, idxh_hbm, idxf_hbm,
               gh_hbm, gf_hbm, s_hbm,
               idx0_v, idxh_v, idxf_v, r_a, r_b, r0_v, sem_a, sem_b, sem0):
    wid = lax.axis_index("s") * NC + lax.axis_index("c")

    pltpu.sync_copy(idx0_hbm.at[wid], idx0_v)
    pltpu.async_copy(e0_hbm.at[idx0_v.at[0]], r0_v, sem0).wait()
    pltpu.sync_copy(r0_v, s_hbm.at[pl.ds(wid * S_PER_W, S_PER_W)])

    pltpu.sync_copy(idxh_hbm.at[wid], idxh_v)
    base_h = wid * (NCH_H * CH_H)

    def body_h(k, carry):
        j0 = 2 * k
        j1 = j0 + 1
        c0 = pltpu.async_copy(e1_hbm.at[idxh_v.at[j0]], r_a, sem_a)
        c1 = pltpu.async_copy(e1_hbm.at[idxh_v.at[j1]], r_b, sem_b)
        c0.wait()
        pltpu.sync_copy(r_a, gh_hbm.at[pl.ds(base_h + j0 * CH_H, CH_H), pl.ds(D, D)])
        c1.wait()
        pltpu.sync_copy(r_b, gh_hbm.at[pl.ds(base_h + j1 * CH_H, CH_H), pl.ds(D, D)])
        return carry

    lax.fori_loop(0, NCH_H // 2, body_h, 0)

    pltpu.sync_copy(idxf_hbm.at[wid], idxf_v)
    base_f = wid * (NCH_F * CH_F)

    def body_f(k, carry):
        j0 = 2 * k
        j1 = j0 + 1
        ra = r_a.at[pl.ds(0, CH_F)]
        rb = r_b.at[pl.ds(0, CH_F)]
        c0 = pltpu.async_copy(e1_hbm.at[idxf_v.at[j0]], ra, sem_a)
        c1 = pltpu.async_copy(e1_hbm.at[idxf_v.at[j1]], rb, sem_b)
        c0.wait()
        pltpu.sync_copy(ra, gf_hbm.at[pl.ds(base_f + j0 * CH_F, CH_F), pl.ds(0, D)])
        c1.wait()
        pltpu.sync_copy(rb, gf_hbm.at[pl.ds(base_f + j1 * CH_F, CH_F), pl.ds(0, D)])
        return carry

    lax.fori_loop(0, NCH_F // 2, body_f, 0)


def _tc_body(xh_ref, xf_ref, gh_ref, gf_ref, wh_ref, bh_ref, wf_ref, bf_ref,
             hist_ref, fut_ref):
    gh = gh_ref[...]
    gf = gf_ref[...]
    low = lax.broadcasted_iota(jnp.int32, (1, 2 * D), 1) < D
    xh = xh_ref[...]
    xf = xf_ref[...]

    def d(x, i, wref, bref):
        return x[:, i:i + 1] * wref[i][None, :] + bref[i][None, :]

    hist_ref[:, 0:128] = jnp.where(low, d(xh, 7, wh_ref, bh_ref), gh)
    hist_ref[:, 128:256] = d(xh, 5, wh_ref, bh_ref) + d(xh, 6, wh_ref, bh_ref)
    hist_ref[:, 256:384] = d(xh, 2, wh_ref, bh_ref) + d(xh, 3, wh_ref, bh_ref)
    hist_ref[:, 384:448] = d(xh, 4, wh_ref, bh_ref)[:, 0:64]
    fut_ref[:, 0:128] = jnp.where(low, gf, d(xf, 5, wf_ref, bf_ref))
    fut_ref[:, 128:192] = d(xf, 6, wf_ref, bf_ref)[:, 0:64]


def kernel(inputs, E0, E1, W2, b2, W3, b3, W4, b4, W5, b5, W6, b6, W7, b7):
    f32 = jnp.float32
    idx0 = inputs[:, 0, 0].astype(jnp.int32).reshape(NW, 1, S_PER_W)
    idxh = inputs[:, :HIST, 1].astype(jnp.int32).reshape(NW, NCH_H, CH_H)
    idxf = inputs[:, HIST:, 1].astype(jnp.int32).reshape(NW, NCH_F, CH_F)

    mesh = plsc.VectorSubcoreMesh(core_axis_name="c", subcore_axis_name="s")
    sc = pl.kernel(
        _sc_gather,
        mesh=mesh,
        out_type=[
            jax.ShapeDtypeStruct((B * HIST, 2 * D), f32),
            jax.ShapeDtypeStruct((B * FUT, 2 * D), f32),
            jax.ShapeDtypeStruct((B, D), f32),
        ],
        scratch_types=[
            pltpu.VMEM((1, S_PER_W), jnp.int32),
            pltpu.VMEM((NCH_H, CH_H), jnp.int32),
            pltpu.VMEM((NCH_F, CH_F), jnp.int32),
            pltpu.VMEM((CH_H, D), f32),
            pltpu.VMEM((CH_H, D), f32),
            pltpu.VMEM((S_PER_W, D), f32),
            pltpu.SemaphoreType.DMA,
            pltpu.SemaphoreType.DMA,
            pltpu.SemaphoreType.DMA,
        ],
        compiler_params=pltpu.CompilerParams(use_tc_tiling_on_sc=False),
    )
    gh, gf, s_rows = sc(E0, E1, idx0, idxh, idxf)

    xh_arr = inputs[:, :HIST, :].reshape(B * HIST, NUM_INPUTS)
    xf_arr = inputs[:, HIST:, :].reshape(B * FUT, NUM_INPUTS)
    z64 = jnp.zeros((D,), f32)
    z128 = jnp.zeros((2 * D,), f32)

    def lo(v):
        return jnp.concatenate([v.reshape(D), z64])

    def hi(v):
        return jnp.concatenate([z64, v.reshape(D)])

    wh = jnp.stack([z128, z128, lo(W2), hi(W3), lo(W4), lo(W5), hi(W6), lo(W7)])
    bh = jnp.stack([z128, z128, lo(b2), hi(b3), lo(b4), lo(b5), hi(b6), lo(b7)])
    wf = jnp.stack([z128, z128, z128, z128, z128, hi(W5), lo(W6), z128])
    bf = jnp.stack([z128, z128, z128, z128, z128, hi(b5), lo(b6), z128])

    bb = 8
    bh_rows = bb * HIST
    bf_rows = bb * FUT
    hist_flat, fut_flat = pl.pallas_call(
        _tc_body,
        grid=(B // bb,),
        in_specs=[
            pl.BlockSpec((bh_rows, NUM_INPUTS), lambda b: (b, 0)),
            pl.BlockSpec((bf_rows, NUM_INPUTS), lambda b: (b, 0)),
            pl.BlockSpec((bh_rows, 2 * D), lambda b: (b, 0)),
            pl.BlockSpec((bf_rows, 2 * D), lambda b: (b, 0)),
            pl.BlockSpec((NUM_INPUTS, 2 * D), lambda b: (0, 0)),
            pl.BlockSpec((NUM_INPUTS, 2 * D), lambda b: (0, 0)),
            pl.BlockSpec((NUM_INPUTS, 2 * D), lambda b: (0, 0)),
            pl.BlockSpec((NUM_INPUTS, 2 * D), lambda b: (0, 0)),
        ],
        out_specs=[
            pl.BlockSpec((bh_rows, 7 * D), lambda b: (b, 0)),
            pl.BlockSpec((bf_rows, 3 * D), lambda b: (b, 0)),
        ],
        out_shape=[
            jax.ShapeDtypeStruct((B * HIST, 7 * D), f32),
            jax.ShapeDtypeStruct((B * FUT, 3 * D), f32),
        ],
    )(xh_arr, xf_arr, gh, gf, wh, bh, wf, bf)

    static = s_rows.reshape(B, 1, D)
    historical = hist_flat.reshape(B, HIST, 7, D)
    future = fut_flat.reshape(B, FUT, 3, D)
    return (static, historical, future)
